# Initial kernel scaffold; baseline (speedup 1.0000x reference)
import jax
import jax.numpy as jnp
from jax.experimental import pallas as pl


def kernel(z, edge_index):
    raise NotImplementedError("write your pallas kernel here")


# speedup vs baseline: 4.0249x; 4.0249x over previous
import functools

import jax
import jax.numpy as jnp
from jax import lax
from jax.experimental import pallas as pl
from jax.experimental.pallas import tpu as pltpu
from jax.experimental.pallas import tpu_sc as plsc

E = 320000
D = 128
NC = 2
NS = 16
NW = NC * NS
EPW = E // NW
C = 400
NCHUNK = EPW // C

_mesh = plsc.VectorSubcoreMesh(core_axis_name="c", subcore_axis_name="s")


@functools.partial(
    pl.kernel,
    out_type=jax.ShapeDtypeStruct((E,), jnp.float32),
    mesh=_mesh,
    scratch_types=[
        pltpu.VMEM((C,), jnp.int32),
        pltpu.VMEM((C,), jnp.int32),
        pltpu.VMEM((C, D), jnp.float32),
        pltpu.VMEM((C, D), jnp.float32),
        pltpu.VMEM((C,), jnp.float32),
        pltpu.SemaphoreType.DMA,
    ],
    compiler_params=pltpu.CompilerParams(needs_layout_passes=False),
)
def _ip_decode(z_hbm, row_hbm, col_hbm, out_hbm,
               ridx_v, cidx_v, arow_v, brow_v, o_v, sem):
    wid = lax.axis_index("s") * NC + lax.axis_index("c")
    ebase = wid * EPW

    def chunk_body(ci, carry):
        base = ebase + ci * C
        pltpu.sync_copy(row_hbm.at[pl.ds(base, C)], ridx_v)
        pltpu.sync_copy(col_hbm.at[pl.ds(base, C)], cidx_v)
        pltpu.async_copy(z_hbm.at[ridx_v], arow_v, sem).wait()
        pltpu.async_copy(z_hbm.at[cidx_v], brow_v, sem).wait()

        lane = lax.iota(jnp.int32, 16)

        def group_body(g, carry2):
            res = jnp.zeros((16,), jnp.float32)
            for j in range(16):
                e = g * 16 + j
                acc = jnp.zeros((16,), jnp.float32)
                for k in range(D // 16):
                    acc = acc + (arow_v[e, pl.ds(k * 16, 16)] *
                                 brow_v[e, pl.ds(k * 16, 16)])
                res = jnp.where(lane == j, jnp.sum(acc), res)
            o_v[pl.ds(g * 16, 16)] = res
            return carry2

        lax.fori_loop(0, C // 16, group_body, 0)
        pltpu.sync_copy(o_v, out_hbm.at[pl.ds(base, C)])
        return carry

    lax.fori_loop(0, NCHUNK, chunk_body, 0)


def kernel(z, edge_index):
    row = edge_index[0].astype(jnp.int32)
    col = edge_index[1].astype(jnp.int32)
    return _ip_decode(z, row, col)
